# Initial kernel scaffold; baseline (speedup 1.0000x reference)
import jax
import jax.numpy as jnp
from jax.experimental import pallas as pl


def kernel(coordinates, hash_table, glf_planes, glf_W, glf_b, fuse_W, fuse_b):
    raise NotImplementedError("write your pallas kernel here")


# speedup vs baseline: 3.1396x; 3.1396x over previous
import functools

import jax
import jax.numpy as jnp
import numpy as np
from jax import lax
from jax.experimental import pallas as pl
from jax.experimental.pallas import tpu as pltpu
from jax.experimental.pallas import tpu_sc as plsc

N_LEVELS = 16
F_PER_LEVEL = 2
LOG2_T = 19
T = 2 ** LOG2_T
BASE_RES = 16
PER_LEVEL_SCALE = 1.3819128800392151
GLF_RES = 64
GLF_RANK = 8
GLF_CH = 8
OUT_DIM = 32
HASH_DIM = N_LEVELS * F_PER_LEVEL
BETA = 0.05
N_PTS = 262144

_P2 = np.int32(np.uint32(2654435761).astype(np.int64) - (1 << 32))
_P3 = np.int32(805459861)
_RES = [int(np.floor(BASE_RES * (PER_LEVEL_SCALE ** l))) for l in range(N_LEVELS)]

NW = 32
PTS_PER_W = N_PTS // NW
CHUNK = 2048
GROUP = 16
N_CHUNKS = PTS_PER_W // CHUNK
GROUPS_PER_CHUNK = CHUNK // GROUP
FEAT = HASH_DIM + 3 * GLF_RANK


def _sc_encoder(xs, ys, zs, table_flat, planes_flat):
    mesh = plsc.VectorSubcoreMesh(core_axis_name="c", subcore_axis_name="s")

    @functools.partial(
        pl.kernel,
        out_type=jax.ShapeDtypeStruct((N_PTS, FEAT), jnp.float32),
        mesh=mesh,
        scratch_types=[
            pltpu.VMEM((3 * GLF_RES * GLF_RES * GLF_RANK,), jnp.float32),
            pltpu.VMEM((CHUNK,), jnp.float32),
            pltpu.VMEM((CHUNK,), jnp.float32),
            pltpu.VMEM((CHUNK,), jnp.float32),
            pltpu.VMEM((2 * N_LEVELS, 8 * GROUP), jnp.int32),
            pltpu.VMEM((2 * N_LEVELS, 8 * GROUP), jnp.float32),
            pltpu.VMEM((GROUP, FEAT), jnp.float32),
            pltpu.SemaphoreType.DMA,
        ],
        compiler_params=pltpu.CompilerParams(needs_layout_passes=False),
    )
    def enc(xs_hbm, ys_hbm, zs_hbm, table_hbm, planes_hbm, feats_hbm,
            planes_v, cx, cy, cz, idx_buf, rows_buf, stage, sem):
        wid = lax.axis_index("s") * 2 + lax.axis_index("c")
        base_pt = wid * PTS_PER_W
        pltpu.sync_copy(planes_hbm, planes_v)

        @pl.loop(0, N_CHUNKS)
        def _chunk(mc):
            cbase = base_pt + mc * CHUNK
            pltpu.sync_copy(xs_hbm.at[pl.ds(cbase, CHUNK)], cx)
            pltpu.sync_copy(ys_hbm.at[pl.ds(cbase, CHUNK)], cy)
            pltpu.sync_copy(zs_hbm.at[pl.ds(cbase, CHUNK)], cz)

            @pl.loop(0, GROUPS_PER_CHUNK)
            def _group(g):
                it = lax.iota(jnp.int32, 16)
                p0 = g * GROUP
                xv = cx[pl.ds(p0, GROUP)]
                yv = cy[pl.ds(p0, GROUP)]
                zv = cz[pl.ds(p0, GROUP)]
                x5 = (xv + 1.0) * 0.5
                y5 = (yv + 1.0) * 0.5
                z5 = (zv + 1.0) * 0.5
                hi = np.float32(1.0 - 1e-6)
                xh = jnp.clip(x5, 0.0, hi)
                yh = jnp.clip(y5, 0.0, hi)
                zh = jnp.clip(z5, 0.0, hi)

                for l in range(N_LEVELS):
                    res = np.float32(_RES[l])
                    ix = (xh * res).astype(jnp.int32)
                    iy = (yh * res).astype(jnp.int32)
                    iz = (zh * res).astype(jnp.int32)
                    hy0 = iy * _P2
                    hy1 = hy0 + _P2
                    hz0 = iz * _P3
                    hz1 = hz0 + _P3
                    mask = np.int32(T - 1)
                    loff = np.int32(l * T)
                    cidx = 0
                    for hz in (hz0, hz1):
                        for hy in (hy0, hy1):
                            tyz = hy ^ hz
                            for hx in (ix, ix + 1):
                                h = ((hx ^ tyz) & mask) | loff
                                h2 = h + h
                                idx_buf[2 * l, pl.ds(cidx * GROUP, GROUP)] = h2
                                idx_buf[2 * l + 1, pl.ds(cidx * GROUP, GROUP)] = h2 + 1
                                cidx += 1
                    pltpu.async_copy(
                        table_hbm.at[idx_buf.at[2 * l]], rows_buf.at[2 * l],
                        sem)
                    pltpu.async_copy(
                        table_hbm.at[idx_buf.at[2 * l + 1]],
                        rows_buf.at[2 * l + 1], sem)

                xg = jnp.clip(x5, 0.0, 1.0) * np.float32(GLF_RES - 1)
                yg = jnp.clip(y5, 0.0, 1.0) * np.float32(GLF_RES - 1)
                zg = jnp.clip(z5, 0.0, 1.0) * np.float32(GLF_RES - 1)
                for p, (u, v) in enumerate(((xg, yg), (xg, zg), (yg, zg))):
                    u0 = u.astype(jnp.int32)
                    v0 = v.astype(jnp.int32)
                    fu = u - u0.astype(jnp.float32)
                    fv = v - v0.astype(jnp.float32)
                    u1 = jnp.minimum(u0 + 1, GLF_RES - 1)
                    v1 = jnp.minimum(v0 + 1, GLF_RES - 1)
                    gu = 1.0 - fu
                    gv = 1.0 - fv
                    pb = np.int32(p * GLF_RES * GLF_RES)
                    b00 = (u0 * GLF_RES + v0 + pb) * GLF_RANK
                    b01 = (u0 * GLF_RES + v1 + pb) * GLF_RANK
                    b10 = (u1 * GLF_RES + v0 + pb) * GLF_RANK
                    b11 = (u1 * GLF_RES + v1 + pb) * GLF_RANK
                    w00 = gu * gv
                    w01 = gu * fv
                    w10 = fu * gv
                    w11 = fu * fv
                    for r in range(GLF_RANK):
                        rr = np.int32(r)
                        acc = w00 * plsc.load_gather(planes_v, [b00 + rr])
                        acc += w01 * plsc.load_gather(planes_v, [b01 + rr])
                        acc += w10 * plsc.load_gather(planes_v, [b10 + rr])
                        acc += w11 * plsc.load_gather(planes_v, [b11 + rr])
                        col = jnp.full((16,), HASH_DIM + p * GLF_RANK + r,
                                       jnp.int32)
                        plsc.store_scatter(stage, [it, col], acc)

                for l in range(2 * N_LEVELS):
                    pltpu.make_async_copy(
                        table_hbm.at[idx_buf.at[l]], rows_buf.at[l], sem
                    ).wait()

                for l in range(N_LEVELS):
                    res = np.float32(_RES[l])
                    px = xh * res
                    py = yh * res
                    pz = zh * res
                    ix = px.astype(jnp.int32)
                    iy = py.astype(jnp.int32)
                    iz = pz.astype(jnp.int32)
                    fx = px - ix.astype(jnp.float32)
                    fy = py - iy.astype(jnp.float32)
                    fz = pz - iz.astype(jnp.float32)
                    gx = 1.0 - fx
                    gy = 1.0 - fy
                    gz = 1.0 - fz
                    wyz = (gy * gz, fy * gz, gy * fz, fy * fz)
                    acc0 = jnp.zeros((16,), jnp.float32)
                    acc1 = jnp.zeros((16,), jnp.float32)
                    r0 = rows_buf.at[2 * l]
                    r1 = rows_buf.at[2 * l + 1]
                    cidx = 0
                    for czi in range(2):
                        for cyi in range(2):
                            wv = wyz[czi * 2 + cyi]
                            for wx in (gx, fx):
                                w = wx * wv
                                ridx = it + np.int32(cidx * GROUP)
                                f0 = plsc.load_gather(r0, [ridx])
                                f1 = plsc.load_gather(r1, [ridx])
                                acc0 += w * f0
                                acc1 += w * f1
                                cidx += 1
                    col0 = jnp.full((16,), 2 * l, jnp.int32)
                    plsc.store_scatter(stage, [it, col0], acc0)
                    plsc.store_scatter(stage, [it, col0 + 1], acc1)

                pltpu.sync_copy(
                    stage, feats_hbm.at[pl.ds(cbase + p0, GROUP), :])

    return enc(xs, ys, zs, table_flat, planes_flat)


def _tc_fuse(feats_fm, M, c):
    BN = 512
    grid = (N_PTS // BN,)

    def body(f_ref, m_ref, c_ref, o_ref):
        f = f_ref[...]
        m = m_ref[...]
        acc = lax.dot_general(
            f, m, (((1,), (0,)), ((), ())),
            preferred_element_type=jnp.float32)
        acc = acc + c_ref[...]
        o_ref[...] = jnp.clip(acc, -1e6, 1e6)

    return pl.pallas_call(
        body,
        grid=grid,
        in_specs=[
            pl.BlockSpec((BN, FEAT), lambda i: (i, 0)),
            pl.BlockSpec((FEAT, OUT_DIM), lambda i: (0, 0)),
            pl.BlockSpec((1, OUT_DIM), lambda i: (0, 0)),
        ],
        out_specs=pl.BlockSpec((BN, OUT_DIM), lambda i: (i, 0)),
        out_shape=jax.ShapeDtypeStruct((N_PTS, OUT_DIM), jnp.float32),
    )(feats_fm, M, c)


def kernel(coordinates, hash_table, glf_planes, glf_W, glf_b, fuse_W, fuse_b):
    xs = coordinates[:, 0]
    ys = coordinates[:, 1]
    zs = coordinates[:, 2]
    table_flat = hash_table.reshape(N_LEVELS * T * F_PER_LEVEL)
    planes_flat = glf_planes.reshape(-1)
    feats = _sc_encoder(xs, ys, zs, table_flat, planes_flat)
    wf_glf = fuse_W[:, HASH_DIM:].T
    M = jnp.concatenate(
        [fuse_W[:, :HASH_DIM].T, BETA * (glf_W @ wf_glf)], axis=0)
    c = (fuse_b + BETA * (glf_b @ wf_glf))[None, :]
    return _tc_fuse(feats, M, c)


# speedup vs baseline: 12.5151x; 3.9862x over previous
import functools

import jax
import jax.numpy as jnp
import numpy as np
from jax import lax
from jax.experimental import pallas as pl
from jax.experimental.pallas import tpu as pltpu
from jax.experimental.pallas import tpu_sc as plsc

N_LEVELS = 16
F_PER_LEVEL = 2
LOG2_T = 19
T = 2 ** LOG2_T
BASE_RES = 16
PER_LEVEL_SCALE = 1.3819128800392151
GLF_RES = 64
GLF_RANK = 8
GLF_CH = 8
OUT_DIM = 32
HASH_DIM = N_LEVELS * F_PER_LEVEL
BETA = 0.05
N_PTS = 262144

_P2 = np.int32(np.uint32(2654435761).astype(np.int64) - (1 << 32))
_P3 = np.int32(805459861)
_RES = [int(np.floor(BASE_RES * (PER_LEVEL_SCALE ** l))) for l in range(N_LEVELS)]

NW = 32
PTS_PER_W = N_PTS // NW
CHUNK = 2048
GROUP = 16
N_CHUNKS = PTS_PER_W // CHUNK
GROUPS_PER_CHUNK = CHUNK // GROUP
FEAT = HASH_DIM + 3 * GLF_RANK


def _sc_encoder(xs, ys, zs, table0, table1, planes_flat):
    mesh = plsc.VectorSubcoreMesh(core_axis_name="c", subcore_axis_name="s")

    @functools.partial(
        pl.kernel,
        out_type=jax.ShapeDtypeStruct((N_PTS, FEAT), jnp.float32),
        mesh=mesh,
        scratch_types=[
            pltpu.VMEM((3 * GLF_RES * GLF_RES * GLF_RANK,), jnp.float32),
            pltpu.VMEM((CHUNK,), jnp.float32),
            pltpu.VMEM((CHUNK,), jnp.float32),
            pltpu.VMEM((CHUNK,), jnp.float32),
            pltpu.VMEM((N_LEVELS, 8 * GROUP), jnp.int32),
            pltpu.VMEM((2 * N_LEVELS, 8 * GROUP), jnp.float32),
            pltpu.VMEM((GROUP, FEAT), jnp.float32),
            pltpu.SemaphoreType.DMA,
        ],
        compiler_params=pltpu.CompilerParams(needs_layout_passes=False),
    )
    def enc(xs_hbm, ys_hbm, zs_hbm, t0_hbm, t1_hbm, planes_hbm, feats_hbm,
            planes_v, cx, cy, cz, idx_buf, rows_buf, stage, sem):
        wid = lax.axis_index("s") * 2 + lax.axis_index("c")
        base_pt = wid * PTS_PER_W
        pltpu.sync_copy(planes_hbm, planes_v)

        @pl.loop(0, N_CHUNKS)
        def _chunk(mc):
            cbase = base_pt + mc * CHUNK
            pltpu.sync_copy(xs_hbm.at[pl.ds(cbase, CHUNK)], cx)
            pltpu.sync_copy(ys_hbm.at[pl.ds(cbase, CHUNK)], cy)
            pltpu.sync_copy(zs_hbm.at[pl.ds(cbase, CHUNK)], cz)

            @pl.loop(0, GROUPS_PER_CHUNK)
            def _group(g):
                it = lax.iota(jnp.int32, 16)
                p0 = g * GROUP
                xv = cx[pl.ds(p0, GROUP)]
                yv = cy[pl.ds(p0, GROUP)]
                zv = cz[pl.ds(p0, GROUP)]
                x5 = (xv + 1.0) * 0.5
                y5 = (yv + 1.0) * 0.5
                z5 = (zv + 1.0) * 0.5
                hi = np.float32(1.0 - 1e-6)
                xh = jnp.clip(x5, 0.0, hi)
                yh = jnp.clip(y5, 0.0, hi)
                zh = jnp.clip(z5, 0.0, hi)

                for l in range(N_LEVELS):
                    res = np.float32(_RES[l])
                    ix = (xh * res).astype(jnp.int32)
                    iy = (yh * res).astype(jnp.int32)
                    iz = (zh * res).astype(jnp.int32)
                    hy0 = iy * _P2
                    hy1 = hy0 + _P2
                    hz0 = iz * _P3
                    hz1 = hz0 + _P3
                    mask = np.int32(T - 1)
                    loff = np.int32(l * T)
                    cidx = 0
                    for hz in (hz0, hz1):
                        for hy in (hy0, hy1):
                            tyz = hy ^ hz
                            for hx in (ix, ix + 1):
                                h = ((hx ^ tyz) & mask) | loff
                                idx_buf[l, pl.ds(cidx * GROUP, GROUP)] = h
                                cidx += 1
                    pltpu.async_copy(
                        t0_hbm.at[idx_buf.at[l]], rows_buf.at[2 * l], sem)
                    pltpu.async_copy(
                        t1_hbm.at[idx_buf.at[l]], rows_buf.at[2 * l + 1], sem)

                xg = jnp.clip(x5, 0.0, 1.0) * np.float32(GLF_RES - 1)
                yg = jnp.clip(y5, 0.0, 1.0) * np.float32(GLF_RES - 1)
                zg = jnp.clip(z5, 0.0, 1.0) * np.float32(GLF_RES - 1)
                for p, (u, v) in enumerate(((xg, yg), (xg, zg), (yg, zg))):
                    u0 = u.astype(jnp.int32)
                    v0 = v.astype(jnp.int32)
                    fu = u - u0.astype(jnp.float32)
                    fv = v - v0.astype(jnp.float32)
                    u1 = jnp.minimum(u0 + 1, GLF_RES - 1)
                    v1 = jnp.minimum(v0 + 1, GLF_RES - 1)
                    gu = 1.0 - fu
                    gv = 1.0 - fv
                    pb = np.int32(p * GLF_RES * GLF_RES)
                    b00 = (u0 * GLF_RES + v0 + pb) * GLF_RANK
                    b01 = (u0 * GLF_RES + v1 + pb) * GLF_RANK
                    b10 = (u1 * GLF_RES + v0 + pb) * GLF_RANK
                    b11 = (u1 * GLF_RES + v1 + pb) * GLF_RANK
                    w00 = gu * gv
                    w01 = gu * fv
                    w10 = fu * gv
                    w11 = fu * fv
                    for r in range(GLF_RANK):
                        rr = np.int32(r)
                        acc = w00 * plsc.load_gather(planes_v, [b00 + rr])
                        acc += w01 * plsc.load_gather(planes_v, [b01 + rr])
                        acc += w10 * plsc.load_gather(planes_v, [b10 + rr])
                        acc += w11 * plsc.load_gather(planes_v, [b11 + rr])
                        col = jnp.full((16,), HASH_DIM + p * GLF_RANK + r,
                                       jnp.int32)
                        plsc.store_scatter(stage, [it, col], acc)

                for l in range(N_LEVELS):
                    pltpu.make_async_copy(
                        t0_hbm.at[idx_buf.at[l]], rows_buf.at[2 * l], sem
                    ).wait()
                    pltpu.make_async_copy(
                        t1_hbm.at[idx_buf.at[l]], rows_buf.at[2 * l + 1], sem
                    ).wait()

                for l in range(N_LEVELS):
                    res = np.float32(_RES[l])
                    px = xh * res
                    py = yh * res
                    pz = zh * res
                    ix = px.astype(jnp.int32)
                    iy = py.astype(jnp.int32)
                    iz = pz.astype(jnp.int32)
                    fx = px - ix.astype(jnp.float32)
                    fy = py - iy.astype(jnp.float32)
                    fz = pz - iz.astype(jnp.float32)
                    gx = 1.0 - fx
                    gy = 1.0 - fy
                    gz = 1.0 - fz
                    wyz = (gy * gz, fy * gz, gy * fz, fy * fz)
                    acc0 = jnp.zeros((16,), jnp.float32)
                    acc1 = jnp.zeros((16,), jnp.float32)
                    r0 = rows_buf.at[2 * l]
                    r1 = rows_buf.at[2 * l + 1]
                    cidx = 0
                    for czi in range(2):
                        for cyi in range(2):
                            wv = wyz[czi * 2 + cyi]
                            for wx in (gx, fx):
                                w = wx * wv
                                ridx = it + np.int32(cidx * GROUP)
                                f0 = plsc.load_gather(r0, [ridx])
                                f1 = plsc.load_gather(r1, [ridx])
                                acc0 += w * f0
                                acc1 += w * f1
                                cidx += 1
                    col0 = jnp.full((16,), 2 * l, jnp.int32)
                    plsc.store_scatter(stage, [it, col0], acc0)
                    plsc.store_scatter(stage, [it, col0 + 1], acc1)

                pltpu.sync_copy(
                    stage, feats_hbm.at[pl.ds(cbase + p0, GROUP), :])

    return enc(xs, ys, zs, table0, table1, planes_flat)


def _tc_fuse(feats_fm, M, c):
    BN = 512
    grid = (N_PTS // BN,)

    def body(f_ref, m_ref, c_ref, o_ref):
        f = f_ref[...]
        m = m_ref[...]
        acc = lax.dot_general(
            f, m, (((1,), (0,)), ((), ())),
            preferred_element_type=jnp.float32)
        acc = acc + c_ref[...]
        o_ref[...] = jnp.clip(acc, -1e6, 1e6)

    return pl.pallas_call(
        body,
        grid=grid,
        in_specs=[
            pl.BlockSpec((BN, FEAT), lambda i: (i, 0)),
            pl.BlockSpec((FEAT, OUT_DIM), lambda i: (0, 0)),
            pl.BlockSpec((1, OUT_DIM), lambda i: (0, 0)),
        ],
        out_specs=pl.BlockSpec((BN, OUT_DIM), lambda i: (i, 0)),
        out_shape=jax.ShapeDtypeStruct((N_PTS, OUT_DIM), jnp.float32),
    )(feats_fm, M, c)


def kernel(coordinates, hash_table, glf_planes, glf_W, glf_b, fuse_W, fuse_b):
    xs = coordinates[:, 0]
    ys = coordinates[:, 1]
    zs = coordinates[:, 2]
    table0 = hash_table[:, :, 0].reshape(N_LEVELS * T)
    table1 = hash_table[:, :, 1].reshape(N_LEVELS * T)
    planes_flat = glf_planes.reshape(-1)
    feats = _sc_encoder(xs, ys, zs, table0, table1, planes_flat)
    wf_glf = fuse_W[:, HASH_DIM:].T
    M = jnp.concatenate(
        [fuse_W[:, :HASH_DIM].T, BETA * (glf_W @ wf_glf)], axis=0)
    c = (fuse_b + BETA * (glf_b @ wf_glf))[None, :]
    return _tc_fuse(feats, M, c)


# speedup vs baseline: 14.7662x; 1.1799x over previous
import functools

import jax
import jax.numpy as jnp
import numpy as np
from jax import lax
from jax.experimental import pallas as pl
from jax.experimental.pallas import tpu as pltpu
from jax.experimental.pallas import tpu_sc as plsc

N_LEVELS = 16
F_PER_LEVEL = 2
LOG2_T = 19
T = 2 ** LOG2_T
BASE_RES = 16
PER_LEVEL_SCALE = 1.3819128800392151
GLF_RES = 64
GLF_RANK = 8
GLF_CH = 8
OUT_DIM = 32
HASH_DIM = N_LEVELS * F_PER_LEVEL
BETA = 0.05
N_PTS = 262144

_P2 = np.int32(np.uint32(2654435761).astype(np.int64) - (1 << 32))
_P3 = np.int32(805459861)
_RES = [int(np.floor(BASE_RES * (PER_LEVEL_SCALE ** l))) for l in range(N_LEVELS)]

NW = 32
PTS_PER_W = N_PTS // NW
CHUNK = 2048
GROUP = 16
N_CHUNKS = PTS_PER_W // CHUNK
GROUPS_PER_CHUNK = CHUNK // GROUP
FEAT = HASH_DIM + 3 * GLF_RANK


def _sc_encoder(xs, ys, zs, table0, table1, planes_flat):
    mesh = plsc.VectorSubcoreMesh(core_axis_name="c", subcore_axis_name="s")

    @functools.partial(
        pl.kernel,
        out_type=jax.ShapeDtypeStruct((N_PTS, FEAT), jnp.float32),
        mesh=mesh,
        scratch_types=[
            pltpu.VMEM((3 * GLF_RES * GLF_RES * GLF_RANK,), jnp.float32),
            pltpu.VMEM((CHUNK,), jnp.float32),
            pltpu.VMEM((CHUNK,), jnp.float32),
            pltpu.VMEM((CHUNK,), jnp.float32),
            pltpu.VMEM((2, N_LEVELS, 8 * GROUP), jnp.int32),
            pltpu.VMEM((2, 2 * N_LEVELS, 8 * GROUP), jnp.float32),
            pltpu.VMEM((GROUP, FEAT), jnp.float32),
            pltpu.SemaphoreType.DMA((2,)),
        ],
        compiler_params=pltpu.CompilerParams(needs_layout_passes=False),
    )
    def enc(xs_hbm, ys_hbm, zs_hbm, t0_hbm, t1_hbm, planes_hbm, feats_hbm,
            planes_v, cx, cy, cz, idx_buf, rows_buf, stage, sem):
        wid = lax.axis_index("s") * 2 + lax.axis_index("c")
        base_pt = wid * PTS_PER_W
        pltpu.sync_copy(planes_hbm, planes_v)

        def load_xyz(g):
            p0 = g * GROUP
            xv = cx[pl.ds(p0, GROUP)]
            yv = cy[pl.ds(p0, GROUP)]
            zv = cz[pl.ds(p0, GROUP)]
            return (xv + 1.0) * 0.5, (yv + 1.0) * 0.5, (zv + 1.0) * 0.5

        def fire(g, par):
            x5, y5, z5 = load_xyz(g)
            hi = np.float32(1.0 - 1e-6)
            xh = jnp.clip(x5, 0.0, hi)
            yh = jnp.clip(y5, 0.0, hi)
            zh = jnp.clip(z5, 0.0, hi)
            ib = idx_buf.at[par]
            rb = rows_buf.at[par]
            sm = sem.at[par]
            for l in range(N_LEVELS):
                res = np.float32(_RES[l])
                ix = (xh * res).astype(jnp.int32)
                iy = (yh * res).astype(jnp.int32)
                iz = (zh * res).astype(jnp.int32)
                hy0 = iy * _P2
                hy1 = hy0 + _P2
                hz0 = iz * _P3
                hz1 = hz0 + _P3
                mask = np.int32(T - 1)
                loff = np.int32(l * T)
                cidx = 0
                for hz in (hz0, hz1):
                    for hy in (hy0, hy1):
                        tyz = hy ^ hz
                        for hx in (ix, ix + 1):
                            h = ((hx ^ tyz) & mask) | loff
                            ib[l, pl.ds(cidx * GROUP, GROUP)] = h
                            cidx += 1
                pltpu.async_copy(t0_hbm.at[ib.at[l]], rb.at[2 * l], sm)
                pltpu.async_copy(t1_hbm.at[ib.at[l]], rb.at[2 * l + 1], sm)

        def consume(g, par, cbase):
            it = lax.iota(jnp.int32, 16)
            x5, y5, z5 = load_xyz(g)
            hi = np.float32(1.0 - 1e-6)
            xh = jnp.clip(x5, 0.0, hi)
            yh = jnp.clip(y5, 0.0, hi)
            zh = jnp.clip(z5, 0.0, hi)

            xg = jnp.clip(x5, 0.0, 1.0) * np.float32(GLF_RES - 1)
            yg = jnp.clip(y5, 0.0, 1.0) * np.float32(GLF_RES - 1)
            zg = jnp.clip(z5, 0.0, 1.0) * np.float32(GLF_RES - 1)
            for p, (u, v) in enumerate(((xg, yg), (xg, zg), (yg, zg))):
                u0 = u.astype(jnp.int32)
                v0 = v.astype(jnp.int32)
                fu = u - u0.astype(jnp.float32)
                fv = v - v0.astype(jnp.float32)
                u1 = jnp.minimum(u0 + 1, GLF_RES - 1)
                v1 = jnp.minimum(v0 + 1, GLF_RES - 1)
                gu = 1.0 - fu
                gv = 1.0 - fv
                pb = np.int32(p * GLF_RES * GLF_RES)
                b00 = (u0 * GLF_RES + v0 + pb) * GLF_RANK
                b01 = (u0 * GLF_RES + v1 + pb) * GLF_RANK
                b10 = (u1 * GLF_RES + v0 + pb) * GLF_RANK
                b11 = (u1 * GLF_RES + v1 + pb) * GLF_RANK
                w00 = gu * gv
                w01 = gu * fv
                w10 = fu * gv
                w11 = fu * fv
                for r in range(GLF_RANK):
                    rr = np.int32(r)
                    acc = w00 * plsc.load_gather(planes_v, [b00 + rr])
                    acc += w01 * plsc.load_gather(planes_v, [b01 + rr])
                    acc += w10 * plsc.load_gather(planes_v, [b10 + rr])
                    acc += w11 * plsc.load_gather(planes_v, [b11 + rr])
                    col = jnp.full((16,), HASH_DIM + p * GLF_RANK + r,
                                   jnp.int32)
                    plsc.store_scatter(stage, [it, col], acc)

            ib = idx_buf.at[par]
            rb = rows_buf.at[par]
            sm = sem.at[par]
            for l in range(N_LEVELS):
                pltpu.make_async_copy(
                    t0_hbm.at[ib.at[l]], rb.at[2 * l], sm).wait()
                pltpu.make_async_copy(
                    t1_hbm.at[ib.at[l]], rb.at[2 * l + 1], sm).wait()

            for l in range(N_LEVELS):
                res = np.float32(_RES[l])
                px = xh * res
                py = yh * res
                pz = zh * res
                ix = px.astype(jnp.int32)
                iy = py.astype(jnp.int32)
                iz = pz.astype(jnp.int32)
                fx = px - ix.astype(jnp.float32)
                fy = py - iy.astype(jnp.float32)
                fz = pz - iz.astype(jnp.float32)
                gx = 1.0 - fx
                gy = 1.0 - fy
                gz = 1.0 - fz
                wyz = (gy * gz, fy * gz, gy * fz, fy * fz)
                acc0 = jnp.zeros((16,), jnp.float32)
                acc1 = jnp.zeros((16,), jnp.float32)
                r0 = rb.at[2 * l]
                r1 = rb.at[2 * l + 1]
                cidx = 0
                for czi in range(2):
                    for cyi in range(2):
                        wv = wyz[czi * 2 + cyi]
                        for wx in (gx, fx):
                            w = wx * wv
                            ridx = it + np.int32(cidx * GROUP)
                            f0 = plsc.load_gather(r0, [ridx])
                            f1 = plsc.load_gather(r1, [ridx])
                            acc0 += w * f0
                            acc1 += w * f1
                            cidx += 1
                col0 = jnp.full((16,), 2 * l, jnp.int32)
                plsc.store_scatter(stage, [it, col0], acc0)
                plsc.store_scatter(stage, [it, col0 + 1], acc1)

            pltpu.sync_copy(
                stage, feats_hbm.at[pl.ds(cbase + g * GROUP, GROUP), :])

        @pl.loop(0, N_CHUNKS)
        def _chunk(mc):
            cbase = base_pt + mc * CHUNK
            pltpu.sync_copy(xs_hbm.at[pl.ds(cbase, CHUNK)], cx)
            pltpu.sync_copy(ys_hbm.at[pl.ds(cbase, CHUNK)], cy)
            pltpu.sync_copy(zs_hbm.at[pl.ds(cbase, CHUNK)], cz)

            fire(0, 0)

            @pl.loop(0, (GROUPS_PER_CHUNK - 2) // 2)
            def _pair(k):
                g = 2 * k
                fire(g + 1, 1)
                consume(g, 0, cbase)
                fire(g + 2, 0)
                consume(g + 1, 1, cbase)

            fire(GROUPS_PER_CHUNK - 1, 1)
            consume(GROUPS_PER_CHUNK - 2, 0, cbase)
            consume(GROUPS_PER_CHUNK - 1, 1, cbase)

    return enc(xs, ys, zs, table0, table1, planes_flat)


def _tc_fuse(feats_fm, M, c):
    BN = 512
    grid = (N_PTS // BN,)

    def body(f_ref, m_ref, c_ref, o_ref):
        f = f_ref[...]
        m = m_ref[...]
        acc = lax.dot_general(
            f, m, (((1,), (0,)), ((), ())),
            preferred_element_type=jnp.float32)
        acc = acc + c_ref[...]
        o_ref[...] = jnp.clip(acc, -1e6, 1e6)

    return pl.pallas_call(
        body,
        grid=grid,
        in_specs=[
            pl.BlockSpec((BN, FEAT), lambda i: (i, 0)),
            pl.BlockSpec((FEAT, OUT_DIM), lambda i: (0, 0)),
            pl.BlockSpec((1, OUT_DIM), lambda i: (0, 0)),
        ],
        out_specs=pl.BlockSpec((BN, OUT_DIM), lambda i: (i, 0)),
        out_shape=jax.ShapeDtypeStruct((N_PTS, OUT_DIM), jnp.float32),
    )(feats_fm, M, c)


def kernel(coordinates, hash_table, glf_planes, glf_W, glf_b, fuse_W, fuse_b):
    xs = coordinates[:, 0]
    ys = coordinates[:, 1]
    zs = coordinates[:, 2]
    table0 = hash_table[:, :, 0].reshape(N_LEVELS * T)
    table1 = hash_table[:, :, 1].reshape(N_LEVELS * T)
    planes_flat = glf_planes.reshape(-1)
    feats = _sc_encoder(xs, ys, zs, table0, table1, planes_flat)
    wf_glf = fuse_W[:, HASH_DIM:].T
    M = jnp.concatenate(
        [fuse_W[:, :HASH_DIM].T, BETA * (glf_W @ wf_glf)], axis=0)
    c = (fuse_b + BETA * (glf_b @ wf_glf))[None, :]
    return _tc_fuse(feats, M, c)


# speedup vs baseline: 23.7755x; 1.6101x over previous
import functools

import jax
import jax.numpy as jnp
import numpy as np
from jax import lax
from jax.experimental import pallas as pl
from jax.experimental.pallas import tpu as pltpu
from jax.experimental.pallas import tpu_sc as plsc

N_LEVELS = 16
F_PER_LEVEL = 2
LOG2_T = 19
T = 2 ** LOG2_T
BASE_RES = 16
PER_LEVEL_SCALE = 1.3819128800392151
GLF_RES = 64
GLF_RANK = 8
GLF_CH = 8
OUT_DIM = 32
HASH_DIM = N_LEVELS * F_PER_LEVEL
BETA = 0.05
N_PTS = 262144

_P2 = np.int32(np.uint32(2654435761).astype(np.int64) - (1 << 32))
_P3 = np.int32(805459861)
_RES = [int(np.floor(BASE_RES * (PER_LEVEL_SCALE ** l))) for l in range(N_LEVELS)]

NW = 32
PTS_PER_W = N_PTS // NW
CHUNK = 2048
GROUP = 16
N_CHUNKS = PTS_PER_W // CHUNK
GROUPS_PER_CHUNK = CHUNK // GROUP
FEAT = HASH_DIM + 3 * GLF_RANK


def _sc_encoder(xs, ys, zs, table_pk, planes_flat):
    mesh = plsc.VectorSubcoreMesh(core_axis_name="c", subcore_axis_name="s")

    @functools.partial(
        pl.kernel,
        out_type=jax.ShapeDtypeStruct((N_PTS, FEAT), jnp.float32),
        mesh=mesh,
        scratch_types=[
            pltpu.VMEM((3 * GLF_RES * GLF_RES * GLF_RANK,), jnp.float32),
            pltpu.VMEM((CHUNK,), jnp.float32),
            pltpu.VMEM((CHUNK,), jnp.float32),
            pltpu.VMEM((CHUNK,), jnp.float32),
            pltpu.VMEM((2, N_LEVELS, 8 * GROUP), jnp.int32),
            pltpu.VMEM((2, N_LEVELS, 8 * GROUP), jnp.int32),
            pltpu.VMEM((GROUP, FEAT), jnp.float32),
            pltpu.SemaphoreType.DMA((2,)),
        ],
        compiler_params=pltpu.CompilerParams(needs_layout_passes=False),
    )
    def enc(xs_hbm, ys_hbm, zs_hbm, tpk_hbm, planes_hbm, feats_hbm,
            planes_v, cx, cy, cz, idx_buf, rows_buf, stage, sem):
        wid = lax.axis_index("s") * 2 + lax.axis_index("c")
        base_pt = wid * PTS_PER_W
        pltpu.sync_copy(planes_hbm, planes_v)

        def load_xyz(g):
            p0 = g * GROUP
            xv = cx[pl.ds(p0, GROUP)]
            yv = cy[pl.ds(p0, GROUP)]
            zv = cz[pl.ds(p0, GROUP)]
            return (xv + 1.0) * 0.5, (yv + 1.0) * 0.5, (zv + 1.0) * 0.5

        def fire(g, par):
            x5, y5, z5 = load_xyz(g)
            hi = np.float32(1.0 - 1e-6)
            xh = jnp.clip(x5, 0.0, hi)
            yh = jnp.clip(y5, 0.0, hi)
            zh = jnp.clip(z5, 0.0, hi)
            ib = idx_buf.at[par]
            rb = rows_buf.at[par]
            sm = sem.at[par]
            for l in range(N_LEVELS):
                res = np.float32(_RES[l])
                ix = (xh * res).astype(jnp.int32)
                iy = (yh * res).astype(jnp.int32)
                iz = (zh * res).astype(jnp.int32)
                hy0 = iy * _P2
                hy1 = hy0 + _P2
                hz0 = iz * _P3
                hz1 = hz0 + _P3
                mask = np.int32(T - 1)
                loff = np.int32(l * T)
                cidx = 0
                for hz in (hz0, hz1):
                    for hy in (hy0, hy1):
                        tyz = hy ^ hz
                        for hx in (ix, ix + 1):
                            h = ((hx ^ tyz) & mask) | loff
                            ib[l, pl.ds(cidx * GROUP, GROUP)] = h
                            cidx += 1
                pltpu.async_copy(tpk_hbm.at[ib.at[l]], rb.at[l], sm)

        def consume(g, par, cbase):
            it = lax.iota(jnp.int32, 16)
            x5, y5, z5 = load_xyz(g)
            hi = np.float32(1.0 - 1e-6)
            xh = jnp.clip(x5, 0.0, hi)
            yh = jnp.clip(y5, 0.0, hi)
            zh = jnp.clip(z5, 0.0, hi)

            xg = jnp.clip(x5, 0.0, 1.0) * np.float32(GLF_RES - 1)
            yg = jnp.clip(y5, 0.0, 1.0) * np.float32(GLF_RES - 1)
            zg = jnp.clip(z5, 0.0, 1.0) * np.float32(GLF_RES - 1)
            for p, (u, v) in enumerate(((xg, yg), (xg, zg), (yg, zg))):
                u0 = u.astype(jnp.int32)
                v0 = v.astype(jnp.int32)
                fu = u - u0.astype(jnp.float32)
                fv = v - v0.astype(jnp.float32)
                u1 = jnp.minimum(u0 + 1, GLF_RES - 1)
                v1 = jnp.minimum(v0 + 1, GLF_RES - 1)
                gu = 1.0 - fu
                gv = 1.0 - fv
                pb = np.int32(p * GLF_RES * GLF_RES)
                b00 = (u0 * GLF_RES + v0 + pb) * GLF_RANK
                b01 = (u0 * GLF_RES + v1 + pb) * GLF_RANK
                b10 = (u1 * GLF_RES + v0 + pb) * GLF_RANK
                b11 = (u1 * GLF_RES + v1 + pb) * GLF_RANK
                w00 = gu * gv
                w01 = gu * fv
                w10 = fu * gv
                w11 = fu * fv
                for r in range(GLF_RANK):
                    rr = np.int32(r)
                    acc = w00 * plsc.load_gather(planes_v, [b00 + rr])
                    acc += w01 * plsc.load_gather(planes_v, [b01 + rr])
                    acc += w10 * plsc.load_gather(planes_v, [b10 + rr])
                    acc += w11 * plsc.load_gather(planes_v, [b11 + rr])
                    col = jnp.full((16,), HASH_DIM + p * GLF_RANK + r,
                                   jnp.int32)
                    plsc.store_scatter(stage, [it, col], acc)

            ib = idx_buf.at[par]
            rb = rows_buf.at[par]
            sm = sem.at[par]
            for l in range(N_LEVELS):
                pltpu.make_async_copy(
                    tpk_hbm.at[ib.at[l]], rb.at[l], sm).wait()

            for l in range(N_LEVELS):
                res = np.float32(_RES[l])
                px = xh * res
                py = yh * res
                pz = zh * res
                ix = px.astype(jnp.int32)
                iy = py.astype(jnp.int32)
                iz = pz.astype(jnp.int32)
                fx = px - ix.astype(jnp.float32)
                fy = py - iy.astype(jnp.float32)
                fz = pz - iz.astype(jnp.float32)
                gx = 1.0 - fx
                gy = 1.0 - fy
                gz = 1.0 - fz
                wyz = (gy * gz, fy * gz, gy * fz, fy * fz)
                acc0 = jnp.zeros((16,), jnp.float32)
                acc1 = jnp.zeros((16,), jnp.float32)
                rl = rb.at[l]
                himask = np.int32(np.uint32(0xFFFF0000).astype(np.int64)
                                  - (1 << 32))
                cidx = 0
                for czi in range(2):
                    for cyi in range(2):
                        wv = wyz[czi * 2 + cyi]
                        for wx in (gx, fx):
                            w = wx * wv
                            ridx = it + np.int32(cidx * GROUP)
                            pkv = plsc.load_gather(rl, [ridx])
                            f0 = plsc.bitcast(
                                lax.shift_left(pkv, 16), jnp.float32)
                            f1 = plsc.bitcast(pkv & himask, jnp.float32)
                            acc0 += w * f0
                            acc1 += w * f1
                            cidx += 1
                col0 = jnp.full((16,), 2 * l, jnp.int32)
                plsc.store_scatter(stage, [it, col0], acc0)
                plsc.store_scatter(stage, [it, col0 + 1], acc1)

            pltpu.sync_copy(
                stage, feats_hbm.at[pl.ds(cbase + g * GROUP, GROUP), :])

        @pl.loop(0, N_CHUNKS)
        def _chunk(mc):
            cbase = base_pt + mc * CHUNK
            pltpu.sync_copy(xs_hbm.at[pl.ds(cbase, CHUNK)], cx)
            pltpu.sync_copy(ys_hbm.at[pl.ds(cbase, CHUNK)], cy)
            pltpu.sync_copy(zs_hbm.at[pl.ds(cbase, CHUNK)], cz)

            fire(0, 0)

            @pl.loop(0, (GROUPS_PER_CHUNK - 2) // 2)
            def _pair(k):
                g = 2 * k
                fire(g + 1, 1)
                consume(g, 0, cbase)
                fire(g + 2, 0)
                consume(g + 1, 1, cbase)

            fire(GROUPS_PER_CHUNK - 1, 1)
            consume(GROUPS_PER_CHUNK - 2, 0, cbase)
            consume(GROUPS_PER_CHUNK - 1, 1, cbase)

    return enc(xs, ys, zs, table_pk, planes_flat)


def _tc_fuse(feats_fm, M, c):
    BN = 512
    grid = (N_PTS // BN,)

    def body(f_ref, m_ref, c_ref, o_ref):
        f = f_ref[...]
        m = m_ref[...]
        acc = lax.dot_general(
            f, m, (((1,), (0,)), ((), ())),
            preferred_element_type=jnp.float32)
        acc = acc + c_ref[...]
        o_ref[...] = jnp.clip(acc, -1e6, 1e6)

    return pl.pallas_call(
        body,
        grid=grid,
        in_specs=[
            pl.BlockSpec((BN, FEAT), lambda i: (i, 0)),
            pl.BlockSpec((FEAT, OUT_DIM), lambda i: (0, 0)),
            pl.BlockSpec((1, OUT_DIM), lambda i: (0, 0)),
        ],
        out_specs=pl.BlockSpec((BN, OUT_DIM), lambda i: (i, 0)),
        out_shape=jax.ShapeDtypeStruct((N_PTS, OUT_DIM), jnp.float32),
    )(feats_fm, M, c)


def kernel(coordinates, hash_table, glf_planes, glf_W, glf_b, fuse_W, fuse_b):
    xs = coordinates[:, 0]
    ys = coordinates[:, 1]
    zs = coordinates[:, 2]
    tb = jax.lax.bitcast_convert_type(
        hash_table.astype(jnp.bfloat16), jnp.uint16).astype(jnp.uint32)
    table_pk = jax.lax.bitcast_convert_type(
        tb[:, :, 0] | (tb[:, :, 1] << 16), jnp.int32).reshape(N_LEVELS * T)
    planes_flat = glf_planes.reshape(-1)
    feats = _sc_encoder(xs, ys, zs, table_pk, planes_flat)
    wf_glf = fuse_W[:, HASH_DIM:].T
    M = jnp.concatenate(
        [fuse_W[:, :HASH_DIM].T, BETA * (glf_W @ wf_glf)], axis=0)
    c = (fuse_b + BETA * (glf_b @ wf_glf))[None, :]
    return _tc_fuse(feats, M, c)


# speedup vs baseline: 23.8000x; 1.0010x over previous
import functools

import jax
import jax.numpy as jnp
import numpy as np
from jax import lax
from jax.experimental import pallas as pl
from jax.experimental.pallas import tpu as pltpu
from jax.experimental.pallas import tpu_sc as plsc

N_LEVELS = 16
F_PER_LEVEL = 2
LOG2_T = 19
T = 2 ** LOG2_T
BASE_RES = 16
PER_LEVEL_SCALE = 1.3819128800392151
GLF_RES = 64
GLF_RANK = 8
GLF_CH = 8
OUT_DIM = 32
HASH_DIM = N_LEVELS * F_PER_LEVEL
BETA = 0.05
N_PTS = 262144

_P2 = np.int32(np.uint32(2654435761).astype(np.int64) - (1 << 32))
_P3 = np.int32(805459861)
_RES = [int(np.floor(BASE_RES * (PER_LEVEL_SCALE ** l))) for l in range(N_LEVELS)]

NW = 32
PTS_PER_W = N_PTS // NW
CHUNK = 2048
GROUP = 16
N_CHUNKS = PTS_PER_W // CHUNK
GROUPS_PER_CHUNK = CHUNK // GROUP
FEAT = HASH_DIM + 3 * GLF_RANK


def _sc_encoder(xs, ys, zs, table_pk, planes_flat):
    mesh = plsc.VectorSubcoreMesh(core_axis_name="c", subcore_axis_name="s")

    @functools.partial(
        pl.kernel,
        out_type=jax.ShapeDtypeStruct((N_PTS, FEAT), jnp.float32),
        mesh=mesh,
        scratch_types=[
            pltpu.VMEM((3 * GLF_RES * GLF_RES * GLF_RANK,), jnp.float32),
            pltpu.VMEM((CHUNK,), jnp.float32),
            pltpu.VMEM((CHUNK,), jnp.float32),
            pltpu.VMEM((CHUNK,), jnp.float32),
            pltpu.VMEM((2, N_LEVELS, 8 * GROUP), jnp.int32),
            pltpu.VMEM((2, N_LEVELS, 8 * GROUP), jnp.int32),
            pltpu.VMEM((2, GROUP, FEAT), jnp.float32),
            pltpu.SemaphoreType.DMA((2,)),
            pltpu.SemaphoreType.DMA((2,)),
        ],
        compiler_params=pltpu.CompilerParams(needs_layout_passes=False),
    )
    def enc(xs_hbm, ys_hbm, zs_hbm, tpk_hbm, planes_hbm, feats_hbm,
            planes_v, cx, cy, cz, idx_buf, rows_buf, stage2, sem, sem_s):
        wid = lax.axis_index("s") * 2 + lax.axis_index("c")
        base_pt = wid * PTS_PER_W
        pltpu.sync_copy(planes_hbm, planes_v)

        def load_xyz(g):
            p0 = g * GROUP
            xv = cx[pl.ds(p0, GROUP)]
            yv = cy[pl.ds(p0, GROUP)]
            zv = cz[pl.ds(p0, GROUP)]
            return (xv + 1.0) * 0.5, (yv + 1.0) * 0.5, (zv + 1.0) * 0.5

        def fire(g, par):
            x5, y5, z5 = load_xyz(g)
            hi = np.float32(1.0 - 1e-6)
            xh = jnp.clip(x5, 0.0, hi)
            yh = jnp.clip(y5, 0.0, hi)
            zh = jnp.clip(z5, 0.0, hi)
            ib = idx_buf.at[par]
            rb = rows_buf.at[par]
            sm = sem.at[par]
            for l in range(N_LEVELS):
                res = np.float32(_RES[l])
                ix = (xh * res).astype(jnp.int32)
                iy = (yh * res).astype(jnp.int32)
                iz = (zh * res).astype(jnp.int32)
                hy0 = iy * _P2
                hy1 = hy0 + _P2
                hz0 = iz * _P3
                hz1 = hz0 + _P3
                mask = np.int32(T - 1)
                loff = np.int32(l * T)
                cidx = 0
                for hz in (hz0, hz1):
                    for hy in (hy0, hy1):
                        tyz = hy ^ hz
                        for hx in (ix, ix + 1):
                            h = ((hx ^ tyz) & mask) | loff
                            ib[l, pl.ds(cidx * GROUP, GROUP)] = h
                            cidx += 1
                pltpu.async_copy(tpk_hbm.at[ib.at[l]], rb.at[l], sm)

        def consume(g, par, cbase, gg):
            it = lax.iota(jnp.int32, 16)
            stage = stage2.at[par]
            dst = feats_hbm.at[pl.ds(cbase + g * GROUP, GROUP), :]

            @pl.when(gg >= 2)
            def _reclaim():
                pltpu.make_async_copy(stage, dst, sem_s.at[par]).wait()
            x5, y5, z5 = load_xyz(g)
            hi = np.float32(1.0 - 1e-6)
            xh = jnp.clip(x5, 0.0, hi)
            yh = jnp.clip(y5, 0.0, hi)
            zh = jnp.clip(z5, 0.0, hi)

            xg = jnp.clip(x5, 0.0, 1.0) * np.float32(GLF_RES - 1)
            yg = jnp.clip(y5, 0.0, 1.0) * np.float32(GLF_RES - 1)
            zg = jnp.clip(z5, 0.0, 1.0) * np.float32(GLF_RES - 1)
            for p, (u, v) in enumerate(((xg, yg), (xg, zg), (yg, zg))):
                u0 = u.astype(jnp.int32)
                v0 = v.astype(jnp.int32)
                fu = u - u0.astype(jnp.float32)
                fv = v - v0.astype(jnp.float32)
                u1 = jnp.minimum(u0 + 1, GLF_RES - 1)
                v1 = jnp.minimum(v0 + 1, GLF_RES - 1)
                gu = 1.0 - fu
                gv = 1.0 - fv
                pb = np.int32(p * GLF_RES * GLF_RES)
                b00 = (u0 * GLF_RES + v0 + pb) * GLF_RANK
                b01 = (u0 * GLF_RES + v1 + pb) * GLF_RANK
                b10 = (u1 * GLF_RES + v0 + pb) * GLF_RANK
                b11 = (u1 * GLF_RES + v1 + pb) * GLF_RANK
                w00 = gu * gv
                w01 = gu * fv
                w10 = fu * gv
                w11 = fu * fv
                for r in range(GLF_RANK):
                    rr = np.int32(r)
                    acc = w00 * plsc.load_gather(planes_v, [b00 + rr])
                    acc += w01 * plsc.load_gather(planes_v, [b01 + rr])
                    acc += w10 * plsc.load_gather(planes_v, [b10 + rr])
                    acc += w11 * plsc.load_gather(planes_v, [b11 + rr])
                    col = jnp.full((16,), HASH_DIM + p * GLF_RANK + r,
                                   jnp.int32)
                    plsc.store_scatter(stage, [it, col], acc)

            ib = idx_buf.at[par]
            rb = rows_buf.at[par]
            sm = sem.at[par]
            for l in range(N_LEVELS):
                pltpu.make_async_copy(
                    tpk_hbm.at[ib.at[l]], rb.at[l], sm).wait()

            for l in range(N_LEVELS):
                res = np.float32(_RES[l])
                px = xh * res
                py = yh * res
                pz = zh * res
                ix = px.astype(jnp.int32)
                iy = py.astype(jnp.int32)
                iz = pz.astype(jnp.int32)
                fx = px - ix.astype(jnp.float32)
                fy = py - iy.astype(jnp.float32)
                fz = pz - iz.astype(jnp.float32)
                gx = 1.0 - fx
                gy = 1.0 - fy
                gz = 1.0 - fz
                wyz = (gy * gz, fy * gz, gy * fz, fy * fz)
                acc0 = jnp.zeros((16,), jnp.float32)
                acc1 = jnp.zeros((16,), jnp.float32)
                rl = rb.at[l]
                himask = np.int32(np.uint32(0xFFFF0000).astype(np.int64)
                                  - (1 << 32))
                cidx = 0
                for czi in range(2):
                    for cyi in range(2):
                        wv = wyz[czi * 2 + cyi]
                        for wx in (gx, fx):
                            w = wx * wv
                            ridx = it + np.int32(cidx * GROUP)
                            pkv = plsc.load_gather(rl, [ridx])
                            f0 = plsc.bitcast(
                                lax.shift_left(pkv, 16), jnp.float32)
                            f1 = plsc.bitcast(pkv & himask, jnp.float32)
                            acc0 += w * f0
                            acc1 += w * f1
                            cidx += 1
                col0 = jnp.full((16,), 2 * l, jnp.int32)
                plsc.store_scatter(stage, [it, col0], acc0)
                plsc.store_scatter(stage, [it, col0 + 1], acc1)

            pltpu.async_copy(stage, dst, sem_s.at[par])

        @pl.loop(0, N_CHUNKS)
        def _chunk(mc):
            cbase = base_pt + mc * CHUNK
            gbase = mc * GROUPS_PER_CHUNK
            pltpu.sync_copy(xs_hbm.at[pl.ds(cbase, CHUNK)], cx)
            pltpu.sync_copy(ys_hbm.at[pl.ds(cbase, CHUNK)], cy)
            pltpu.sync_copy(zs_hbm.at[pl.ds(cbase, CHUNK)], cz)

            fire(0, 0)

            @pl.loop(0, (GROUPS_PER_CHUNK - 2) // 2)
            def _pair(k):
                g = 2 * k
                fire(g + 1, 1)
                consume(g, 0, cbase, gbase + g)
                fire(g + 2, 0)
                consume(g + 1, 1, cbase, gbase + g + 1)

            fire(GROUPS_PER_CHUNK - 1, 1)
            consume(GROUPS_PER_CHUNK - 2, 0, cbase,
                    gbase + GROUPS_PER_CHUNK - 2)
            consume(GROUPS_PER_CHUNK - 1, 1, cbase,
                    gbase + GROUPS_PER_CHUNK - 1)

        for par in range(2):
            pltpu.make_async_copy(
                stage2.at[par],
                feats_hbm.at[pl.ds(base_pt, GROUP), :],
                sem_s.at[par]).wait()

    return enc(xs, ys, zs, table_pk, planes_flat)


def _tc_fuse(feats_fm, M, c):
    BN = 512
    grid = (N_PTS // BN,)

    def body(f_ref, m_ref, c_ref, o_ref):
        f = f_ref[...]
        m = m_ref[...]
        acc = lax.dot_general(
            f, m, (((1,), (0,)), ((), ())),
            preferred_element_type=jnp.float32)
        acc = acc + c_ref[...]
        o_ref[...] = jnp.clip(acc, -1e6, 1e6)

    return pl.pallas_call(
        body,
        grid=grid,
        in_specs=[
            pl.BlockSpec((BN, FEAT), lambda i: (i, 0)),
            pl.BlockSpec((FEAT, OUT_DIM), lambda i: (0, 0)),
            pl.BlockSpec((1, OUT_DIM), lambda i: (0, 0)),
        ],
        out_specs=pl.BlockSpec((BN, OUT_DIM), lambda i: (i, 0)),
        out_shape=jax.ShapeDtypeStruct((N_PTS, OUT_DIM), jnp.float32),
    )(feats_fm, M, c)


def kernel(coordinates, hash_table, glf_planes, glf_W, glf_b, fuse_W, fuse_b):
    xs = coordinates[:, 0]
    ys = coordinates[:, 1]
    zs = coordinates[:, 2]
    tb = jax.lax.bitcast_convert_type(
        hash_table.astype(jnp.bfloat16), jnp.uint16).astype(jnp.uint32)
    table_pk = jax.lax.bitcast_convert_type(
        tb[:, :, 0] | (tb[:, :, 1] << 16), jnp.int32).reshape(N_LEVELS * T)
    planes_flat = glf_planes.reshape(-1)
    feats = _sc_encoder(xs, ys, zs, table_pk, planes_flat)
    wf_glf = fuse_W[:, HASH_DIM:].T
    M = jnp.concatenate(
        [fuse_W[:, :HASH_DIM].T, BETA * (glf_W @ wf_glf)], axis=0)
    c = (fuse_b + BETA * (glf_b @ wf_glf))[None, :]
    return _tc_fuse(feats, M, c)


# speedup vs baseline: 27.4591x; 1.1537x over previous
import functools

import jax
import jax.numpy as jnp
import numpy as np
from jax import lax
from jax.experimental import pallas as pl
from jax.experimental.pallas import tpu as pltpu
from jax.experimental.pallas import tpu_sc as plsc

N_LEVELS = 16
F_PER_LEVEL = 2
LOG2_T = 19
T = 2 ** LOG2_T
BASE_RES = 16
PER_LEVEL_SCALE = 1.3819128800392151
GLF_RES = 64
GLF_RANK = 8
GLF_CH = 8
OUT_DIM = 32
HASH_DIM = N_LEVELS * F_PER_LEVEL
BETA = 0.05
N_PTS = 262144

_P2 = np.int32(np.uint32(2654435761).astype(np.int64) - (1 << 32))
_P3 = np.int32(805459861)
_RES = [int(np.floor(BASE_RES * (PER_LEVEL_SCALE ** l))) for l in range(N_LEVELS)]

NW = 32
PTS_PER_W = N_PTS // NW
CHUNK = 2048
GROUP = 16
N_CHUNKS = PTS_PER_W // CHUNK
GROUPS_PER_CHUNK = CHUNK // GROUP
FEAT = HASH_DIM + 3 * GLF_RANK


def _sc_encoder(xs, ys, zs, table_pk, planes_flat):
    mesh = plsc.VectorSubcoreMesh(core_axis_name="c", subcore_axis_name="s")

    @functools.partial(
        pl.kernel,
        out_type=jax.ShapeDtypeStruct((N_PTS, FEAT), jnp.float32),
        mesh=mesh,
        scratch_types=[
            pltpu.VMEM((3 * GLF_RES * GLF_RES * GLF_RANK,), jnp.float32),
            pltpu.VMEM((CHUNK,), jnp.float32),
            pltpu.VMEM((CHUNK,), jnp.float32),
            pltpu.VMEM((CHUNK,), jnp.float32),
            pltpu.VMEM((2, N_LEVELS, 8 * GROUP), jnp.int32),
            pltpu.VMEM((2, N_LEVELS, 8 * GROUP), jnp.int32),
            pltpu.VMEM((2, GROUP, FEAT), jnp.float32),
            pltpu.SemaphoreType.DMA((2,)),
            pltpu.SemaphoreType.DMA((2,)),
        ],
        compiler_params=pltpu.CompilerParams(needs_layout_passes=False),
    )
    def enc(xs_hbm, ys_hbm, zs_hbm, tpk_hbm, planes_hbm, feats_hbm,
            planes_v, cx, cy, cz, idx_buf, rows_buf, stage2, sem, sem_s):
        wid = lax.axis_index("s") * 2 + lax.axis_index("c")
        base_pt = wid * PTS_PER_W
        pltpu.sync_copy(planes_hbm, planes_v)

        def load_xyz(g):
            p0 = g * GROUP
            xv = cx[pl.ds(p0, GROUP)]
            yv = cy[pl.ds(p0, GROUP)]
            zv = cz[pl.ds(p0, GROUP)]
            return (xv + 1.0) * 0.5, (yv + 1.0) * 0.5, (zv + 1.0) * 0.5

        def fire(g, par):
            x5, y5, z5 = load_xyz(g)
            hi = np.float32(1.0 - 1e-6)
            xh = jnp.clip(x5, 0.0, hi)
            yh = jnp.clip(y5, 0.0, hi)
            zh = jnp.clip(z5, 0.0, hi)
            ib = idx_buf.at[par]
            rb = rows_buf.at[par]
            sm = sem.at[par]
            for l in range(N_LEVELS):
                res = np.float32(_RES[l])
                ix = (xh * res).astype(jnp.int32)
                iy = (yh * res).astype(jnp.int32)
                iz = (zh * res).astype(jnp.int32)
                hy0 = iy * _P2
                hy1 = hy0 + _P2
                hz0 = iz * _P3
                hz1 = hz0 + _P3
                mask = np.int32(T - 1)
                loff = np.int32(l * T)
                cidx = 0
                for hz in (hz0, hz1):
                    for hy in (hy0, hy1):
                        tyz = hy ^ hz
                        for hx in (ix, ix + 1):
                            h = ((hx ^ tyz) & mask) | loff
                            ib[l, pl.ds(cidx * GROUP, GROUP)] = h
                            cidx += 1
                pltpu.async_copy(tpk_hbm.at[ib.at[l]], rb.at[l], sm)

        def consume(g, par, cbase, gg):
            it = lax.iota(jnp.int32, 16)
            stage = stage2.at[par]
            dst = feats_hbm.at[pl.ds(cbase + g * GROUP, GROUP), :]

            @pl.when(gg >= 2)
            def _reclaim():
                pltpu.make_async_copy(stage, dst, sem_s.at[par]).wait()
            x5, y5, z5 = load_xyz(g)
            hi = np.float32(1.0 - 1e-6)
            xh = jnp.clip(x5, 0.0, hi)
            yh = jnp.clip(y5, 0.0, hi)
            zh = jnp.clip(z5, 0.0, hi)

            xg = jnp.clip(x5, 0.0, 1.0) * np.float32(GLF_RES - 1)
            yg = jnp.clip(y5, 0.0, 1.0) * np.float32(GLF_RES - 1)
            zg = jnp.clip(z5, 0.0, 1.0) * np.float32(GLF_RES - 1)
            for p, (u, v) in enumerate(((xg, yg), (xg, zg), (yg, zg))):
                u0 = u.astype(jnp.int32)
                v0 = v.astype(jnp.int32)
                fu = u - u0.astype(jnp.float32)
                fv = v - v0.astype(jnp.float32)
                u1 = jnp.minimum(u0 + 1, GLF_RES - 1)
                v1 = jnp.minimum(v0 + 1, GLF_RES - 1)
                gu = 1.0 - fu
                gv = 1.0 - fv
                pb = np.int32(p * GLF_RES * GLF_RES)
                b00 = (u0 * GLF_RES + v0 + pb) * GLF_RANK
                b01 = (u0 * GLF_RES + v1 + pb) * GLF_RANK
                b10 = (u1 * GLF_RES + v0 + pb) * GLF_RANK
                b11 = (u1 * GLF_RES + v1 + pb) * GLF_RANK
                w00 = gu * gv
                w01 = gu * fv
                w10 = fu * gv
                w11 = fu * fv
                for r in range(GLF_RANK):
                    rr = np.int32(r)
                    acc = w00 * plsc.load_gather(planes_v, [b00 + rr])
                    acc += w01 * plsc.load_gather(planes_v, [b01 + rr])
                    acc += w10 * plsc.load_gather(planes_v, [b10 + rr])
                    acc += w11 * plsc.load_gather(planes_v, [b11 + rr])
                    col = jnp.full((16,), HASH_DIM + p * GLF_RANK + r,
                                   jnp.int32)
                    plsc.store_scatter(stage, [it, col], acc)

            ib = idx_buf.at[par]
            rb = rows_buf.at[par]
            sm = sem.at[par]
            for l in range(N_LEVELS):
                pltpu.make_async_copy(
                    tpk_hbm.at[ib.at[l]], rb.at[l], sm).wait()

            for l in range(N_LEVELS):
                res = np.float32(_RES[l])
                px = xh * res
                py = yh * res
                pz = zh * res
                ix = px.astype(jnp.int32)
                iy = py.astype(jnp.int32)
                iz = pz.astype(jnp.int32)
                fx = px - ix.astype(jnp.float32)
                fy = py - iy.astype(jnp.float32)
                fz = pz - iz.astype(jnp.float32)
                gx = 1.0 - fx
                gy = 1.0 - fy
                gz = 1.0 - fz
                wyz = (gy * gz, fy * gz, gy * fz, fy * fz)
                acc0 = jnp.zeros((16,), jnp.float32)
                acc1 = jnp.zeros((16,), jnp.float32)
                rl = rb.at[l]
                himask = np.int32(np.uint32(0xFFFF0000).astype(np.int64)
                                  - (1 << 32))
                cidx = 0
                for czi in range(2):
                    for cyi in range(2):
                        wv = wyz[czi * 2 + cyi]
                        for wx in (gx, fx):
                            w = wx * wv
                            ridx = it + np.int32(cidx * GROUP)
                            pkv = plsc.load_gather(rl, [ridx])
                            f0 = plsc.bitcast(
                                lax.shift_left(pkv, 16), jnp.float32)
                            f1 = plsc.bitcast(pkv & himask, jnp.float32)
                            acc0 += w * f0
                            acc1 += w * f1
                            cidx += 1
                col0 = jnp.full((16,), 2 * l, jnp.int32)
                plsc.store_scatter(stage, [it, col0], acc0)
                plsc.store_scatter(stage, [it, col0 + 1], acc1)

            pltpu.async_copy(stage, dst, sem_s.at[par])

        @pl.loop(0, N_CHUNKS)
        def _chunk(mc):
            cbase = base_pt + mc * CHUNK
            gbase = mc * GROUPS_PER_CHUNK
            pltpu.sync_copy(xs_hbm.at[pl.ds(cbase, CHUNK)], cx)
            pltpu.sync_copy(ys_hbm.at[pl.ds(cbase, CHUNK)], cy)
            pltpu.sync_copy(zs_hbm.at[pl.ds(cbase, CHUNK)], cz)

            fire(0, 0)

            @pl.loop(0, (GROUPS_PER_CHUNK - 2) // 2)
            def _pair(k):
                g = 2 * k
                fire(g + 1, 1)
                consume(g, 0, cbase, gbase + g)
                fire(g + 2, 0)
                consume(g + 1, 1, cbase, gbase + g + 1)

            fire(GROUPS_PER_CHUNK - 1, 1)
            consume(GROUPS_PER_CHUNK - 2, 0, cbase,
                    gbase + GROUPS_PER_CHUNK - 2)
            consume(GROUPS_PER_CHUNK - 1, 1, cbase,
                    gbase + GROUPS_PER_CHUNK - 1)

        for par in range(2):
            pltpu.make_async_copy(
                stage2.at[par],
                feats_hbm.at[pl.ds(base_pt, GROUP), :],
                sem_s.at[par]).wait()

    return enc(xs, ys, zs, table_pk, planes_flat)


def _tc_fuse(feats_fm, M, c):
    BN = 8192
    grid = (N_PTS // BN,)

    def body(f_ref, m_ref, c_ref, o_ref):
        f = f_ref[...]
        m = m_ref[...]
        acc = lax.dot_general(
            f, m, (((1,), (0,)), ((), ())),
            preferred_element_type=jnp.float32)
        acc = acc + c_ref[...]
        o_ref[...] = jnp.clip(acc, -1e6, 1e6)

    return pl.pallas_call(
        body,
        grid=grid,
        in_specs=[
            pl.BlockSpec((BN, FEAT), lambda i: (i, 0)),
            pl.BlockSpec((FEAT, OUT_DIM), lambda i: (0, 0)),
            pl.BlockSpec((1, OUT_DIM), lambda i: (0, 0)),
        ],
        out_specs=pl.BlockSpec((BN, OUT_DIM), lambda i: (i, 0)),
        out_shape=jax.ShapeDtypeStruct((N_PTS, OUT_DIM), jnp.float32),
    )(feats_fm, M, c)


def kernel(coordinates, hash_table, glf_planes, glf_W, glf_b, fuse_W, fuse_b):
    xs = coordinates[:, 0]
    ys = coordinates[:, 1]
    zs = coordinates[:, 2]
    tb = jax.lax.bitcast_convert_type(
        hash_table.astype(jnp.bfloat16), jnp.uint16).astype(jnp.uint32)
    table_pk = jax.lax.bitcast_convert_type(
        tb[:, :, 0] | (tb[:, :, 1] << 16), jnp.int32).reshape(N_LEVELS * T)
    planes_flat = glf_planes.reshape(-1)
    feats = _sc_encoder(xs, ys, zs, table_pk, planes_flat)
    wf_glf = fuse_W[:, HASH_DIM:].T
    M = jnp.concatenate(
        [fuse_W[:, :HASH_DIM].T, BETA * (glf_W @ wf_glf)], axis=0)
    c = (fuse_b + BETA * (glf_b @ wf_glf))[None, :]
    return _tc_fuse(feats, M, c)
